# Initial kernel scaffold; baseline (speedup 1.0000x reference)
import jax
import jax.numpy as jnp
from jax.experimental import pallas as pl


def kernel(x, params, edge_index, batch):
    raise NotImplementedError("write your pallas kernel here")


# speedup vs baseline: 6.5026x; 6.5026x over previous
import functools

import jax
import jax.numpy as jnp
from jax import lax
from jax.experimental import pallas as pl
from jax.experimental.pallas import tpu as pltpu
from jax.experimental.pallas import tpu_sc as plsc

N = 50000
E = 800000
D = 128
H = 64
G = 64

HALF = 25000
ACC_ROWS = 26624
TRASH = 26000
CHUNK = 128
NW = 32
EPAD = 802816
ITERS = EPAD // (16 * CHUNK)
DRAIN_FULL = HALF // CHUNK
DRAIN_REM = HALF - DRAIN_FULL * CHUNK
ZCHUNKS = ACC_ROWS // (16 * CHUNK)

BS = 1000
NB = N // BS


def _make_sc_scatter(width):
  mesh = plsc.VectorSubcoreMesh(core_axis_name="c", subcore_axis_name="s")

  @functools.partial(
      pl.kernel,
      mesh=mesh,
      compiler_params=pltpu.CompilerParams(use_tc_tiling_on_sc=False),
      out_type=jax.ShapeDtypeStruct((N, width), jnp.float32),
      scratch_types=[
          pltpu.VMEM((CHUNK, width), jnp.float32),
          pltpu.VMEM((CHUNK,), jnp.int32),
          pltpu.VMEM((CHUNK,), jnp.int32),
          pltpu.VMEM((CHUNK, width), jnp.float32),
          pltpu.VMEM_SHARED((ACC_ROWS, width), jnp.float32),
          pltpu.SemaphoreType.DMA,
      ],
  )
  def sc_scatter(table_hbm, src_hbm, dstloc_hbm, out_hbm,
                 rows_v, src_v, dst_v, zrows_v, acc_sh, sem):
    c = lax.axis_index("c")
    s = lax.axis_index("s")

    z16 = jnp.zeros((16,), jnp.float32)

    def zrow(i, _):
      for j in range(width // 16):
        zrows_v[i, pl.ds(j * 16, 16)] = z16
      return _

    lax.fori_loop(0, CHUNK, zrow, None)
    for k in range(ZCHUNKS):
      base = (s * ZCHUNKS + k) * CHUNK
      pltpu.sync_copy(zrows_v, acc_sh.at[pl.ds(base, CHUNK)])
    plsc.subcore_barrier()

    def step(i, _):
      ebase = (s * ITERS + i) * CHUNK
      pltpu.sync_copy(src_hbm.at[pl.ds(ebase, CHUNK)], src_v)
      pltpu.sync_copy(dstloc_hbm.at[pl.ds(c * EPAD + ebase, CHUNK)], dst_v)
      pltpu.async_copy(table_hbm.at[src_v], rows_v, sem).wait()
      pltpu.sync_copy(rows_v, acc_sh.at[dst_v], add=True)
      return _

    lax.fori_loop(0, ITERS, step, None)
    plsc.subcore_barrier()

    for j in range(ZCHUNKS):
      cid = s + 16 * j

      @pl.when(cid < DRAIN_FULL)
      def _():
        pltpu.sync_copy(acc_sh.at[pl.ds(cid * CHUNK, CHUNK)],
                        out_hbm.at[pl.ds(c * HALF + cid * CHUNK, CHUNK)])

      @pl.when(cid == DRAIN_FULL)
      def _():
        pltpu.sync_copy(acc_sh.at[pl.ds(DRAIN_FULL * CHUNK, DRAIN_REM)],
                        out_hbm.at[pl.ds(c * HALF + DRAIN_FULL * CHUNK,
                                         DRAIN_REM)])

  return sc_scatter


_sc_scatter64 = _make_sc_scatter(H)
_sc_scatter16 = _make_sc_scatter(16)


def _matmul(h, wt, degcol, scale):
  din = h.shape[1]

  def body(h_ref, wt_ref, d_ref, o_ref):
    acc = jnp.dot(h_ref[...], wt_ref[...], preferred_element_type=jnp.float32)
    if scale:
      o_ref[...] = lax.rsqrt(d_ref[...] + 1.0) * acc
    else:
      o_ref[...] = acc

  return pl.pallas_call(
      body,
      grid=(NB,),
      in_specs=[
          pl.BlockSpec((BS, din), lambda i: (i, 0)),
          pl.BlockSpec((din, H), lambda i: (0, 0)),
          pl.BlockSpec((BS, 1), lambda i: (i, 0)),
      ],
      out_specs=pl.BlockSpec((BS, H), lambda i: (i, 0)),
      out_shape=jax.ShapeDtypeStruct((N, H), jnp.float32),
  )(h, wt, degcol)


def _agg_stats(acc, mp, degcol):
  def body(a_ref, m_ref, d_ref, agg_ref, st_ref):
    i = pl.program_id(0)
    agg = lax.rsqrt(d_ref[...] + 1.0) * (a_ref[...] + m_ref[...])
    agg_ref[...] = agg

    @pl.when(i == 0)
    def _():
      st_ref[...] = jnp.zeros_like(st_ref)

    st_ref[0:1, :] += jnp.sum(agg, axis=0, keepdims=True)
    st_ref[1:2, :] += jnp.sum(agg * agg, axis=0, keepdims=True)

  return pl.pallas_call(
      body,
      grid=(NB,),
      in_specs=[
          pl.BlockSpec((BS, H), lambda i: (i, 0)),
          pl.BlockSpec((BS, H), lambda i: (i, 0)),
          pl.BlockSpec((BS, 1), lambda i: (i, 0)),
      ],
      out_specs=[
          pl.BlockSpec((BS, H), lambda i: (i, 0)),
          pl.BlockSpec((8, H), lambda i: (0, 0)),
      ],
      out_shape=[
          jax.ShapeDtypeStruct((N, H), jnp.float32),
          jax.ShapeDtypeStruct((8, H), jnp.float32),
      ],
  )(acc, mp, degcol)


def _bn_relu_res(agg, stats, ident, g, be):
  def body(a_ref, st_ref, id_ref, g_ref, be_ref, o_ref):
    mu = st_ref[0:1, :] * (1.0 / N)
    ex2 = st_ref[1:2, :] * (1.0 / N)
    var = ex2 - mu * mu
    y = (a_ref[...] - mu) * lax.rsqrt(var + 1e-5)
    o_ref[...] = jnp.maximum(y * g_ref[...] + be_ref[...], 0.0) + id_ref[...]

  return pl.pallas_call(
      body,
      grid=(NB,),
      in_specs=[
          pl.BlockSpec((BS, H), lambda i: (i, 0)),
          pl.BlockSpec((8, H), lambda i: (0, 0)),
          pl.BlockSpec((BS, H), lambda i: (i, 0)),
          pl.BlockSpec((1, H), lambda i: (0, 0)),
          pl.BlockSpec((1, H), lambda i: (0, 0)),
      ],
      out_specs=pl.BlockSpec((BS, H), lambda i: (i, 0)),
      out_shape=jax.ShapeDtypeStruct((N, H), jnp.float32),
  )(agg, stats, ident, g, be)


def _pool(h, batch2, wlt, bl):
  def body(h_ref, b_ref, wlt_ref, bl_ref, o_ref, ps_ref, cs_ref):
    i = pl.program_id(0)

    @pl.when(i == 0)
    def _():
      ps_ref[...] = jnp.zeros_like(ps_ref)
      cs_ref[...] = jnp.zeros_like(cs_ref)

    onehot = (b_ref[...] == lax.broadcasted_iota(jnp.int32, (1, G), 1)
              ).astype(jnp.float32)
    dn = (((0,), (0,)), ((), ()))
    ps_ref[...] += lax.dot_general(onehot, h_ref[...], dn,
                                   preferred_element_type=jnp.float32)
    cs_ref[...] += lax.dot_general(onehot, jnp.ones((BS, 1), jnp.float32), dn,
                                   preferred_element_type=jnp.float32)
    pooled = ps_ref[...] / jnp.maximum(cs_ref[...], 1.0)
    o_ref[...] = jnp.dot(pooled, wlt_ref[...],
                         preferred_element_type=jnp.float32) + bl_ref[...]

  out, _, _ = pl.pallas_call(
      body,
      grid=(NB,),
      in_specs=[
          pl.BlockSpec((BS, H), lambda i: (i, 0)),
          pl.BlockSpec((BS, 1), lambda i: (i, 0)),
          pl.BlockSpec((H, 1), lambda i: (0, 0)),
          pl.BlockSpec((1, 1), lambda i: (0, 0)),
      ],
      out_specs=[
          pl.BlockSpec((G, 1), lambda i: (0, 0)),
          pl.BlockSpec((G, H), lambda i: (0, 0)),
          pl.BlockSpec((G, 1), lambda i: (0, 0)),
      ],
      out_shape=[
          jax.ShapeDtypeStruct((G, 1), jnp.float32),
          jax.ShapeDtypeStruct((G, H), jnp.float32),
          jax.ShapeDtypeStruct((G, 1), jnp.float32),
      ],
  )(h, batch2, wlt, bl)
  return out


def kernel(x, params, edge_index, batch):
  src = edge_index[0]
  dst = edge_index[1]
  pad = EPAD - E
  src_p = jnp.concatenate([src, jnp.zeros((pad,), jnp.int32)])
  trash = jnp.full((pad,), TRASH, jnp.int32)
  dl0 = jnp.concatenate([jnp.where(dst < HALF, dst, TRASH), trash])
  dl1 = jnp.concatenate([jnp.where(dst >= HALF, dst - HALF, TRASH), trash])
  dstloc = jnp.concatenate([dl0, dl1])

  ones16 = jnp.ones((N, 16), jnp.float32)
  deg16 = _sc_scatter16(ones16, src_p, dstloc)
  degcol = deg16[:, :1]

  batch2 = batch[:, None]
  g = {i: params[f'g{i}'][None, :] for i in range(1, 6)}
  be = {i: params[f'be{i}'][None, :] for i in range(1, 6)}

  ident = _matmul(x, params['Wp'].T, degcol, scale=False)
  mp = _matmul(x, params['W1'].T, degcol, scale=True)
  h = x
  for i in range(1, 6):
    acc = _sc_scatter64(mp, src_p, dstloc)
    agg, stats = _agg_stats(acc, mp, degcol)
    h = _bn_relu_res(agg, stats, ident, g[i], be[i])
    ident = h
    if i < 5:
      mp = _matmul(h, params[f'W{i + 1}'].T, degcol, scale=True)

  return _pool(h, batch2, params['Wl'].T, params['bl'][None, :])


# speedup vs baseline: 9.0203x; 1.3872x over previous
import functools

import jax
import jax.numpy as jnp
from jax import lax
from jax.experimental import pallas as pl
from jax.experimental.pallas import tpu as pltpu
from jax.experimental.pallas import tpu_sc as plsc

N = 50000
E = 800000
D = 128
H = 64
G = 64

HALF = 25000
ACC_ROWS = 25088
TRASH = 25080
CHUNK = 128
NW = 32
EPAD = 802816
ITERS = EPAD // (16 * CHUNK)
NCHUNKS = EPAD // CHUNK
KBUF = 3
NFULL = ITERS // KBUF
TAIL = ITERS - NFULL * KBUF
DRAIN_FULL = HALF // CHUNK
DRAIN_REM = HALF - DRAIN_FULL * CHUNK
ZROWS = ACC_ROWS // 16

BS = 1000
NB = N // BS


def _make_sc_scatter(width):
  mesh = plsc.VectorSubcoreMesh(core_axis_name="c", subcore_axis_name="s")

  @functools.partial(
      pl.kernel,
      mesh=mesh,
      compiler_params=pltpu.CompilerParams(use_tc_tiling_on_sc=False),
      out_type=jax.ShapeDtypeStruct((N, width), jnp.float32),
      scratch_types=[
          pltpu.VMEM((KBUF, CHUNK, width), jnp.float32),
          pltpu.VMEM((KBUF, 2, CHUNK), jnp.int32),
          pltpu.VMEM_SHARED((ACC_ROWS, width), jnp.float32),
          pltpu.SemaphoreType.DMA,
          pltpu.SemaphoreType.DMA,
      ],
  )
  def sc_scatter(table_hbm, sdl_hbm, out_hbm,
                 rows_v, sdl_v, acc_sh, sem_i, sem_g):
    c = lax.axis_index("c")
    s = lax.axis_index("s")

    z16 = jnp.zeros((16,), jnp.float32)

    def zrow(i, _):
      for j in range(width // 16):
        rows_v[0, i, pl.ds(j * 16, 16)] = z16
      return _

    lax.fori_loop(0, CHUNK, zrow, None)
    for k in range(ZROWS // CHUNK):
      pltpu.sync_copy(rows_v.at[0],
                      acc_sh.at[pl.ds(s * ZROWS + k * CHUNK, CHUNK)])
    zrem = ZROWS % CHUNK
    if zrem:
      pltpu.sync_copy(rows_v.at[0, pl.ds(0, zrem)],
                      acc_sh.at[pl.ds(s * ZROWS + ZROWS - zrem, zrem)])
    plsc.subcore_barrier()

    base_chunk = c * NCHUNKS + s * ITERS

    def group(first_chunk, k):
      idx_cps = []
      for b in range(k):
        idx_cps.append(pltpu.async_copy(sdl_hbm.at[first_chunk + b],
                                        sdl_v.at[b], sem_i))
      g_cps = []
      for b in range(k):
        idx_cps[b].wait()
        g_cps.append(
            pltpu.async_copy(table_hbm.at[sdl_v.at[b, 0]], rows_v.at[b],
                             sem_g))
      for b in range(k):
        g_cps[b].wait()
        pltpu.sync_copy(rows_v.at[b], acc_sh.at[sdl_v.at[b, 1]], add=True)

    def outer(gi, _):
      group(base_chunk + gi * KBUF, KBUF)
      return _

    lax.fori_loop(0, NFULL, outer, None)
    if TAIL:
      group(base_chunk + NFULL * KBUF, TAIL)
    plsc.subcore_barrier()

    for j in range(13):
      cid = s + 16 * j

      @pl.when(cid < DRAIN_FULL)
      def _():
        pltpu.sync_copy(acc_sh.at[pl.ds(cid * CHUNK, CHUNK)],
                        out_hbm.at[pl.ds(c * HALF + cid * CHUNK, CHUNK)])

      @pl.when(cid == DRAIN_FULL)
      def _():
        pltpu.sync_copy(acc_sh.at[pl.ds(DRAIN_FULL * CHUNK, DRAIN_REM)],
                        out_hbm.at[pl.ds(c * HALF + DRAIN_FULL * CHUNK,
                                         DRAIN_REM)])

  return sc_scatter


_sc_scatter64 = _make_sc_scatter(H)
_sc_scatter16 = _make_sc_scatter(16)


def _matmul(h, wt, degcol, scale):
  din = h.shape[1]

  def body(h_ref, wt_ref, d_ref, o_ref):
    acc = jnp.dot(h_ref[...], wt_ref[...], preferred_element_type=jnp.float32)
    if scale:
      o_ref[...] = lax.rsqrt(d_ref[...] + 1.0) * acc
    else:
      o_ref[...] = acc

  return pl.pallas_call(
      body,
      grid=(NB,),
      in_specs=[
          pl.BlockSpec((BS, din), lambda i: (i, 0)),
          pl.BlockSpec((din, H), lambda i: (0, 0)),
          pl.BlockSpec((BS, 1), lambda i: (i, 0)),
      ],
      out_specs=pl.BlockSpec((BS, H), lambda i: (i, 0)),
      out_shape=jax.ShapeDtypeStruct((N, H), jnp.float32),
  )(h, wt, degcol)


def _agg_stats(acc, mp, degcol):
  def body(a_ref, m_ref, d_ref, agg_ref, st_ref):
    i = pl.program_id(0)
    agg = lax.rsqrt(d_ref[...] + 1.0) * (a_ref[...] + m_ref[...])
    agg_ref[...] = agg

    @pl.when(i == 0)
    def _():
      st_ref[...] = jnp.zeros_like(st_ref)

    st_ref[0:1, :] += jnp.sum(agg, axis=0, keepdims=True)
    st_ref[1:2, :] += jnp.sum(agg * agg, axis=0, keepdims=True)

  return pl.pallas_call(
      body,
      grid=(NB,),
      in_specs=[
          pl.BlockSpec((BS, H), lambda i: (i, 0)),
          pl.BlockSpec((BS, H), lambda i: (i, 0)),
          pl.BlockSpec((BS, 1), lambda i: (i, 0)),
      ],
      out_specs=[
          pl.BlockSpec((BS, H), lambda i: (i, 0)),
          pl.BlockSpec((8, H), lambda i: (0, 0)),
      ],
      out_shape=[
          jax.ShapeDtypeStruct((N, H), jnp.float32),
          jax.ShapeDtypeStruct((8, H), jnp.float32),
      ],
  )(acc, mp, degcol)


def _bn_relu_res(agg, stats, ident, g, be):
  def body(a_ref, st_ref, id_ref, g_ref, be_ref, o_ref):
    mu = st_ref[0:1, :] * (1.0 / N)
    ex2 = st_ref[1:2, :] * (1.0 / N)
    var = ex2 - mu * mu
    y = (a_ref[...] - mu) * lax.rsqrt(var + 1e-5)
    o_ref[...] = jnp.maximum(y * g_ref[...] + be_ref[...], 0.0) + id_ref[...]

  return pl.pallas_call(
      body,
      grid=(NB,),
      in_specs=[
          pl.BlockSpec((BS, H), lambda i: (i, 0)),
          pl.BlockSpec((8, H), lambda i: (0, 0)),
          pl.BlockSpec((BS, H), lambda i: (i, 0)),
          pl.BlockSpec((1, H), lambda i: (0, 0)),
          pl.BlockSpec((1, H), lambda i: (0, 0)),
      ],
      out_specs=pl.BlockSpec((BS, H), lambda i: (i, 0)),
      out_shape=jax.ShapeDtypeStruct((N, H), jnp.float32),
  )(agg, stats, ident, g, be)


def _pool(h, batch2, wlt, bl):
  def body(h_ref, b_ref, wlt_ref, bl_ref, o_ref, ps_ref, cs_ref):
    i = pl.program_id(0)

    @pl.when(i == 0)
    def _():
      ps_ref[...] = jnp.zeros_like(ps_ref)
      cs_ref[...] = jnp.zeros_like(cs_ref)

    onehot = (b_ref[...] == lax.broadcasted_iota(jnp.int32, (1, G), 1)
              ).astype(jnp.float32)
    dn = (((0,), (0,)), ((), ()))
    ps_ref[...] += lax.dot_general(onehot, h_ref[...], dn,
                                   preferred_element_type=jnp.float32)
    cs_ref[...] += lax.dot_general(onehot, jnp.ones((BS, 1), jnp.float32), dn,
                                   preferred_element_type=jnp.float32)
    pooled = ps_ref[...] / jnp.maximum(cs_ref[...], 1.0)
    o_ref[...] = jnp.dot(pooled, wlt_ref[...],
                         preferred_element_type=jnp.float32) + bl_ref[...]

  out, _, _ = pl.pallas_call(
      body,
      grid=(NB,),
      in_specs=[
          pl.BlockSpec((BS, H), lambda i: (i, 0)),
          pl.BlockSpec((BS, 1), lambda i: (i, 0)),
          pl.BlockSpec((H, 1), lambda i: (0, 0)),
          pl.BlockSpec((1, 1), lambda i: (0, 0)),
      ],
      out_specs=[
          pl.BlockSpec((G, 1), lambda i: (0, 0)),
          pl.BlockSpec((G, H), lambda i: (0, 0)),
          pl.BlockSpec((G, 1), lambda i: (0, 0)),
      ],
      out_shape=[
          jax.ShapeDtypeStruct((G, 1), jnp.float32),
          jax.ShapeDtypeStruct((G, H), jnp.float32),
          jax.ShapeDtypeStruct((G, 1), jnp.float32),
      ],
  )(h, batch2, wlt, bl)
  return out


def kernel(x, params, edge_index, batch):
  src = edge_index[0]
  dst = edge_index[1]
  pad = EPAD - E
  src_p = jnp.concatenate([src, jnp.zeros((pad,), jnp.int32)])
  trash = jnp.full((pad,), TRASH, jnp.int32)
  dl0 = jnp.concatenate([jnp.where(dst < HALF, dst, TRASH), trash])
  dl1 = jnp.concatenate([jnp.where(dst >= HALF, dst - HALF, TRASH), trash])
  src_c = src_p.reshape(NCHUNKS, 1, CHUNK)
  sdl = jnp.concatenate([
      jnp.concatenate([src_c, dl0.reshape(NCHUNKS, 1, CHUNK)], axis=1),
      jnp.concatenate([src_c, dl1.reshape(NCHUNKS, 1, CHUNK)], axis=1),
  ], axis=0)

  ones16 = jnp.ones((N, 16), jnp.float32)
  deg16 = _sc_scatter16(ones16, sdl)
  degcol = deg16[:, :1]

  batch2 = batch[:, None]
  g = {i: params[f'g{i}'][None, :] for i in range(1, 6)}
  be = {i: params[f'be{i}'][None, :] for i in range(1, 6)}

  ident = _matmul(x, params['Wp'].T, degcol, scale=False)
  mp = _matmul(x, params['W1'].T, degcol, scale=True)
  h = x
  for i in range(1, 6):
    acc = _sc_scatter64(mp, sdl)
    agg, stats = _agg_stats(acc, mp, degcol)
    h = _bn_relu_res(agg, stats, ident, g[i], be[i])
    ident = h
    if i < 5:
      mp = _matmul(h, params[f'W{i + 1}'].T, degcol, scale=True)

  return _pool(h, batch2, params['Wl'].T, params['bl'][None, :])
